# baseline (device time: 181110 ns/iter reference)
import jax
import jax.numpy as jnp
from jax import lax
from jax.experimental import pallas as pl
from jax.experimental.pallas import tpu as pltpu

N_DEV = 8
B, S, C_LOC, T = 4, 512, 256, 4
OUT_N = 256


def kernel(x, k, Wp):
    def body(x_ref, k_ref, wp_ref, out_ref, comm_ref, send_sems, recv_sems):
        my = lax.axis_index("i")
        left = lax.rem(my + N_DEV - 1, N_DEV)
        right = lax.rem(my + 1, N_DEV)

        barrier_sem = pltpu.get_barrier_semaphore()
        for nbr in (left, right):
            pl.semaphore_signal(
                barrier_sem,
                inc=1,
                device_id=(nbr,),
                device_id_type=pl.DeviceIdType.MESH,
            )
        pl.semaphore_wait(barrier_sem, 2)

        xx = x_ref[...]
        acc = xx * k_ref[T - 1].reshape(1, 1, C_LOC)
        for lag in range(1, T):
            tap = k_ref[T - 1 - lag].reshape(1, 1, C_LOC)
            shifted = jnp.concatenate(
                [jnp.zeros((B, lag, C_LOC), jnp.float32), xx[:, : S - lag, :]],
                axis=1,
            )
            acc = acc + shifted * tap
        a = acc * jax.nn.sigmoid(acc)
        partial = jnp.dot(
            a.reshape(B * S, C_LOC), wp_ref[...],
            preferred_element_type=jnp.float32,
        )

        comm_ref[0] = partial
        total = partial
        for h in range(N_DEV - 1):
            rdma = pltpu.make_async_remote_copy(
                src_ref=comm_ref.at[h],
                dst_ref=comm_ref.at[h + 1],
                send_sem=send_sems.at[h],
                recv_sem=recv_sems.at[h],
                device_id=(right,),
                device_id_type=pl.DeviceIdType.MESH,
            )
            rdma.start()
            rdma.wait()
            total = total + comm_ref[h + 1]

        out_ref[...] = total.reshape(B, S, OUT_N)

    return pl.pallas_call(
        body,
        out_shape=jax.ShapeDtypeStruct((B, S, OUT_N), jnp.float32),
        in_specs=[pl.BlockSpec(memory_space=pltpu.VMEM)] * 3,
        out_specs=pl.BlockSpec(memory_space=pltpu.VMEM),
        scratch_shapes=[
            pltpu.VMEM((N_DEV, B * S, OUT_N), jnp.float32),
            pltpu.SemaphoreType.DMA((N_DEV - 1,)),
            pltpu.SemaphoreType.DMA((N_DEV - 1,)),
        ],
        compiler_params=pltpu.CompilerParams(collective_id=0),
    )(x, k, Wp)


# device time: 58845 ns/iter; 3.0777x vs baseline; 3.0777x over previous
import jax
import jax.numpy as jnp
from jax import lax
from jax.experimental import pallas as pl
from jax.experimental.pallas import tpu as pltpu

N_DEV = 8
B, S, C_LOC, T = 4, 512, 256, 4
OUT_N = 256
M = B * S

_HALVING = [(1024, 0), (512, 1024), (256, 1536)]


def kernel(x, k, Wp):
    def body(x_ref, k_ref, wp_ref, out_ref, work_ref, stage_ref, res_ref,
             send_sems, recv_sems):
        my = lax.axis_index("i")
        p = lax.rem(my, 4)
        bz = my // 4
        bx = jnp.bitwise_xor(p & 1, (p >> 1) & 1)
        by = (p >> 1) & 1
        x_partner = bz * 4 + jnp.bitwise_xor(p, 1)
        y_partner = bz * 4 + jnp.bitwise_xor(p, 3)
        z_partner = jnp.bitwise_xor(my, 4)
        partners = [x_partner, y_partner, z_partner]
        bits = [bx, by, bz]

        barrier_sem = pltpu.get_barrier_semaphore()
        for nbr in partners:
            pl.semaphore_signal(
                barrier_sem,
                inc=1,
                device_id=(nbr,),
                device_id_type=pl.DeviceIdType.MESH,
            )
        pl.semaphore_wait(barrier_sem, 3)

        xx = x_ref[...]
        acc = xx * k_ref[T - 1].reshape(1, 1, C_LOC)
        for lag in range(1, T):
            tap = k_ref[T - 1 - lag].reshape(1, 1, C_LOC)
            shifted = jnp.concatenate(
                [jnp.zeros((B, lag, C_LOC), jnp.float32), xx[:, : S - lag, :]],
                axis=1,
            )
            acc = acc + shifted * tap
        a = acc * jax.nn.sigmoid(acc)
        work_ref[...] = jnp.dot(
            a.reshape(M, C_LOC), wp_ref[...],
            preferred_element_type=jnp.float32,
        )

        base = 0
        for r, (rows, stage_off) in enumerate(_HALVING):
            keep_off = base + bits[r] * rows
            send_off = base + (1 - bits[r]) * rows
            rdma = pltpu.make_async_remote_copy(
                src_ref=work_ref.at[pl.ds(send_off, rows)],
                dst_ref=stage_ref.at[pl.ds(stage_off, rows)],
                send_sem=send_sems.at[r],
                recv_sem=recv_sems.at[r],
                device_id=(partners[r],),
                device_id_type=pl.DeviceIdType.MESH,
            )
            rdma.start()
            rdma.wait()
            work_ref[pl.ds(keep_off, rows)] = (
                work_ref[pl.ds(keep_off, rows)]
                + stage_ref[pl.ds(stage_off, rows)]
            )
            base = keep_off

        res_ref[pl.ds(base, 256)] = work_ref[pl.ds(base, 256)]

        for r in (2, 1, 0):
            rows = _HALVING[r][0]
            rdma = pltpu.make_async_remote_copy(
                src_ref=res_ref.at[pl.ds(base, rows)],
                dst_ref=res_ref.at[pl.ds(base, rows)],
                send_sem=send_sems.at[3 + r],
                recv_sem=recv_sems.at[3 + r],
                device_id=(partners[r],),
                device_id_type=pl.DeviceIdType.MESH,
            )
            rdma.start()
            rdma.wait()
            base = base - bits[r] * rows

        out_ref[...] = res_ref[...].reshape(B, S, OUT_N)

    return pl.pallas_call(
        body,
        out_shape=jax.ShapeDtypeStruct((B, S, OUT_N), jnp.float32),
        in_specs=[pl.BlockSpec(memory_space=pltpu.VMEM)] * 3,
        out_specs=pl.BlockSpec(memory_space=pltpu.VMEM),
        scratch_shapes=[
            pltpu.VMEM((M, OUT_N), jnp.float32),
            pltpu.VMEM((1792, OUT_N), jnp.float32),
            pltpu.VMEM((M, OUT_N), jnp.float32),
            pltpu.SemaphoreType.DMA((6,)),
            pltpu.SemaphoreType.DMA((6,)),
        ],
        compiler_params=pltpu.CompilerParams(collective_id=0),
    )(x, k, Wp)


# device time: 39290 ns/iter; 4.6096x vs baseline; 1.4977x over previous
import jax
import jax.numpy as jnp
from jax import lax
from jax.experimental import pallas as pl
from jax.experimental.pallas import tpu as pltpu

N_DEV = 8
B, S, C_LOC, T = 4, 512, 256, 4
OUT_N = 256
M = B * S

_HALVING = [(1024, 0), (512, 1024), (256, 1536)]


def kernel(x, k, Wp):
    def body(x_ref, k_ref, wp_ref, out_ref, work_ref, sendbuf_ref, stage_ref,
             res_ref, send_sems, recv_sems):
        my = lax.axis_index("i")
        p = lax.rem(my, 4)
        bz = my // 4
        bx = jnp.bitwise_xor(p & 1, (p >> 1) & 1)
        by = (p >> 1) & 1
        x_partner = bz * 4 + jnp.bitwise_xor(p, 1)
        y_partner = bz * 4 + jnp.bitwise_xor(p, 3)
        z_partner = jnp.bitwise_xor(my, 4)
        partners = [x_partner, y_partner, z_partner]
        bits = [bx, by, bz]

        barrier_sem = pltpu.get_barrier_semaphore()
        for nbr in partners:
            pl.semaphore_signal(
                barrier_sem,
                inc=1,
                device_id=(nbr,),
                device_id_type=pl.DeviceIdType.MESH,
            )
        pl.semaphore_wait(barrier_sem, 3)

        xx = x_ref[...]
        acc = xx * k_ref[T - 1].reshape(1, 1, C_LOC)
        for lag in range(1, T):
            tap = k_ref[T - 1 - lag].reshape(1, 1, C_LOC)
            shifted = jnp.concatenate(
                [jnp.zeros((B, lag, C_LOC), jnp.float32), xx[:, : S - lag, :]],
                axis=1,
            )
            acc = acc + shifted * tap
        a = acc * jax.nn.sigmoid(acc)
        work_ref[...] = jnp.dot(
            a.reshape(M, C_LOC).astype(jnp.bfloat16),
            wp_ref[...].astype(jnp.bfloat16),
            preferred_element_type=jnp.float32,
        )

        base = 0
        for r, (rows, stage_off) in enumerate(_HALVING):
            keep_off = base + bits[r] * rows
            send_off = base + (1 - bits[r]) * rows
            sendbuf_ref[pl.ds(stage_off, rows)] = work_ref[
                pl.ds(send_off, rows)
            ].astype(jnp.bfloat16)
            rdma = pltpu.make_async_remote_copy(
                src_ref=sendbuf_ref.at[pl.ds(stage_off, rows)],
                dst_ref=stage_ref.at[pl.ds(stage_off, rows)],
                send_sem=send_sems.at[r],
                recv_sem=recv_sems.at[r],
                device_id=(partners[r],),
                device_id_type=pl.DeviceIdType.MESH,
            )
            rdma.start()
            rdma.wait()
            work_ref[pl.ds(keep_off, rows)] = (
                work_ref[pl.ds(keep_off, rows)]
                + stage_ref[pl.ds(stage_off, rows)].astype(jnp.float32)
            )
            base = keep_off

        res_ref[pl.ds(base, 256)] = work_ref[pl.ds(base, 256)].astype(
            jnp.bfloat16
        )

        for r in (2, 1, 0):
            rows = _HALVING[r][0]
            rdma = pltpu.make_async_remote_copy(
                src_ref=res_ref.at[pl.ds(base, rows)],
                dst_ref=res_ref.at[pl.ds(base, rows)],
                send_sem=send_sems.at[3 + r],
                recv_sem=recv_sems.at[3 + r],
                device_id=(partners[r],),
                device_id_type=pl.DeviceIdType.MESH,
            )
            rdma.start()
            rdma.wait()
            base = base - bits[r] * rows

        out_ref[...] = res_ref[...].astype(jnp.float32).reshape(B, S, OUT_N)

    return pl.pallas_call(
        body,
        out_shape=jax.ShapeDtypeStruct((B, S, OUT_N), jnp.float32),
        in_specs=[pl.BlockSpec(memory_space=pltpu.VMEM)] * 3,
        out_specs=pl.BlockSpec(memory_space=pltpu.VMEM),
        scratch_shapes=[
            pltpu.VMEM((M, OUT_N), jnp.float32),
            pltpu.VMEM((1792, OUT_N), jnp.bfloat16),
            pltpu.VMEM((1792, OUT_N), jnp.bfloat16),
            pltpu.VMEM((M, OUT_N), jnp.bfloat16),
            pltpu.SemaphoreType.DMA((6,)),
            pltpu.SemaphoreType.DMA((6,)),
        ],
        compiler_params=pltpu.CompilerParams(collective_id=0),
    )(x, k, Wp)


# device time: 38586 ns/iter; 4.6937x vs baseline; 1.0182x over previous
import jax
import jax.numpy as jnp
from jax import lax
from jax.experimental import pallas as pl
from jax.experimental.pallas import tpu as pltpu

N_DEV = 8
B, S, C_LOC, T = 4, 512, 256, 4
OUT_N = 256
M = B * S

_HALVING = [(1024, 0), (512, 1024), (256, 1536)]


def kernel(x, k, Wp):
    def body(x_ref, k_ref, wp_ref, out_ref, work_ref, sendbuf_ref, stage_ref,
             res_ref, send_sems, recv_sems):
        my = lax.axis_index("i")
        p = lax.rem(my, 4)
        bz = my // 4
        bx = jnp.bitwise_xor(p & 1, (p >> 1) & 1)
        by = (p >> 1) & 1
        x_partner = bz * 4 + jnp.bitwise_xor(p, 1)
        y_partner = bz * 4 + jnp.bitwise_xor(p, 3)
        z_partner = jnp.bitwise_xor(my, 4)
        partners = [x_partner, y_partner, z_partner]
        bits = [bx, by, bz]

        barrier_sem = pltpu.get_barrier_semaphore()
        for nbr in partners:
            pl.semaphore_signal(
                barrier_sem,
                inc=1,
                device_id=(nbr,),
                device_id_type=pl.DeviceIdType.MESH,
            )
        pl.semaphore_wait(barrier_sem, 3)

        wp_bf16 = wp_ref[...].astype(jnp.bfloat16)

        def compute_half(b_off):
            xx = x_ref[pl.ds(b_off, 2)]
            acc = xx * k_ref[T - 1].reshape(1, 1, C_LOC)
            for lag in range(1, T):
                tap = k_ref[T - 1 - lag].reshape(1, 1, C_LOC)
                shifted = jnp.concatenate(
                    [jnp.zeros((2, lag, C_LOC), jnp.float32),
                     xx[:, : S - lag, :]],
                    axis=1,
                )
                acc = acc + shifted * tap
            a = acc * jax.nn.sigmoid(acc)
            return jnp.dot(
                a.reshape(1024, C_LOC).astype(jnp.bfloat16),
                wp_bf16,
                preferred_element_type=jnp.float32,
            )

        keep_off = bx * 1024
        send_off = (1 - bx) * 1024
        sendbuf_ref[pl.ds(0, 1024)] = compute_half((1 - bx) * 2).astype(
            jnp.bfloat16
        )
        rdma0 = pltpu.make_async_remote_copy(
            src_ref=sendbuf_ref.at[pl.ds(0, 1024)],
            dst_ref=stage_ref.at[pl.ds(0, 1024)],
            send_sem=send_sems.at[0],
            recv_sem=recv_sems.at[0],
            device_id=(partners[0],),
            device_id_type=pl.DeviceIdType.MESH,
        )
        rdma0.start()
        work_ref[pl.ds(keep_off, 1024)] = compute_half(bx * 2)
        rdma0.wait()
        work_ref[pl.ds(keep_off, 1024)] = (
            work_ref[pl.ds(keep_off, 1024)]
            + stage_ref[pl.ds(0, 1024)].astype(jnp.float32)
        )
        base = keep_off

        for r, (rows, stage_off) in list(enumerate(_HALVING))[1:]:
            keep_off = base + bits[r] * rows
            send_off = base + (1 - bits[r]) * rows
            sendbuf_ref[pl.ds(stage_off, rows)] = work_ref[
                pl.ds(send_off, rows)
            ].astype(jnp.bfloat16)
            rdma = pltpu.make_async_remote_copy(
                src_ref=sendbuf_ref.at[pl.ds(stage_off, rows)],
                dst_ref=stage_ref.at[pl.ds(stage_off, rows)],
                send_sem=send_sems.at[r],
                recv_sem=recv_sems.at[r],
                device_id=(partners[r],),
                device_id_type=pl.DeviceIdType.MESH,
            )
            rdma.start()
            rdma.wait()
            work_ref[pl.ds(keep_off, rows)] = (
                work_ref[pl.ds(keep_off, rows)]
                + stage_ref[pl.ds(stage_off, rows)].astype(jnp.float32)
            )
            base = keep_off

        res_ref[pl.ds(base, 256)] = work_ref[pl.ds(base, 256)].astype(
            jnp.bfloat16
        )

        for r in (2, 1, 0):
            rows = _HALVING[r][0]
            rdma = pltpu.make_async_remote_copy(
                src_ref=res_ref.at[pl.ds(base, rows)],
                dst_ref=res_ref.at[pl.ds(base, rows)],
                send_sem=send_sems.at[3 + r],
                recv_sem=recv_sems.at[3 + r],
                device_id=(partners[r],),
                device_id_type=pl.DeviceIdType.MESH,
            )
            rdma.start()
            rdma.wait()
            base = base - bits[r] * rows

        out_ref[...] = res_ref[...].astype(jnp.float32).reshape(B, S, OUT_N)

    return pl.pallas_call(
        body,
        out_shape=jax.ShapeDtypeStruct((B, S, OUT_N), jnp.float32),
        in_specs=[pl.BlockSpec(memory_space=pltpu.VMEM)] * 3,
        out_specs=pl.BlockSpec(memory_space=pltpu.VMEM),
        scratch_shapes=[
            pltpu.VMEM((M, OUT_N), jnp.float32),
            pltpu.VMEM((1792, OUT_N), jnp.bfloat16),
            pltpu.VMEM((1792, OUT_N), jnp.bfloat16),
            pltpu.VMEM((M, OUT_N), jnp.bfloat16),
            pltpu.SemaphoreType.DMA((6,)),
            pltpu.SemaphoreType.DMA((6,)),
        ],
        compiler_params=pltpu.CompilerParams(collective_id=0),
    )(x, k, Wp)


# device time: 28310 ns/iter; 6.3974x vs baseline; 1.3630x over previous
import jax
import jax.numpy as jnp
from jax import lax
from jax.experimental import pallas as pl
from jax.experimental.pallas import tpu as pltpu

N_DEV = 8
B, S, C_LOC, T = 4, 512, 256, 4
OUT_N = 256
M = B * S
CH = 256

_DELTAS = [(1, 0, 0), (1, 1, 0), (1, 0, 1), (1, 1, 1),
           (0, 1, 0), (0, 0, 1), (0, 1, 1)]


def kernel(x, k, Wp):
    def body(x_ref, k_ref, wp_ref, out_ref, sendbuf_ref, stage_ref, res_ref,
             near_ref, rs_send_sems, rs_recv_sems, ag_send_sems,
             ag_recv_sems):
        my = lax.axis_index("i")
        p = lax.rem(my, 4)
        bz = my // 4
        bx = jnp.bitwise_xor(p & 1, (p >> 1) & 1)
        by = (p >> 1) & 1

        def peer(delta):
            dx, dy, dz = delta
            tx = jnp.bitwise_xor(bx, dx)
            ty = jnp.bitwise_xor(by, dy)
            tz = jnp.bitwise_xor(bz, dz)
            dev = tz * 4 + ty * 2 + jnp.bitwise_xor(tx, ty)
            off3 = tx * 1024 + ty * 512 + tz * CH
            return dev, off3

        peers = [peer(d) for d in _DELTAS]
        my_off3 = bx * 1024 + by * 512 + bz * CH

        barrier_sem = pltpu.get_barrier_semaphore()
        for dev, _ in peers:
            pl.semaphore_signal(
                barrier_sem,
                inc=1,
                device_id=(dev,),
                device_id_type=pl.DeviceIdType.MESH,
            )
        pl.semaphore_wait(barrier_sem, N_DEV - 1)

        wp_bf16 = wp_ref[...].astype(jnp.bfloat16)

        def compute_half(b_off):
            xx = x_ref[pl.ds(b_off, 2)]
            acc = xx * k_ref[T - 1].reshape(1, 1, C_LOC)
            for lag in range(1, T):
                tap = k_ref[T - 1 - lag].reshape(1, 1, C_LOC)
                shifted = jnp.concatenate(
                    [jnp.zeros((2, lag, C_LOC), jnp.float32),
                     xx[:, : S - lag, :]],
                    axis=1,
                )
                acc = acc + shifted * tap
            a = acc * jax.nn.sigmoid(acc)
            return jnp.dot(
                a.reshape(1024, C_LOC).astype(jnp.bfloat16),
                wp_bf16,
                preferred_element_type=jnp.float32,
            )

        def rs_send(r):
            dev, off3 = peers[r]
            rdma = pltpu.make_async_remote_copy(
                src_ref=sendbuf_ref.at[pl.ds(off3, CH)],
                dst_ref=stage_ref.at[pl.ds(r * CH, CH)],
                send_sem=rs_send_sems.at[r],
                recv_sem=rs_recv_sems.at[r],
                device_id=(dev,),
                device_id_type=pl.DeviceIdType.MESH,
            )
            rdma.start()
            return rdma

        half_far = compute_half((1 - bx) * 2)
        sendbuf_ref[pl.ds((1 - bx) * 1024, 1024)] = half_far.astype(
            jnp.bfloat16
        )
        sends = [rs_send(r) for r in range(4)]
        near_ref[...] = compute_half(bx * 2)
        sendbuf_ref[pl.ds(bx * 1024, 1024)] = near_ref[...].astype(
            jnp.bfloat16
        )
        sends += [rs_send(r) for r in range(4, 7)]

        own = near_ref[pl.ds(by * 512 + bz * CH, CH)]

        red = own
        for r in range(7):
            recv = pltpu.make_async_remote_copy(
                src_ref=sendbuf_ref.at[pl.ds(0, CH)],
                dst_ref=stage_ref.at[pl.ds(r * CH, CH)],
                send_sem=rs_send_sems.at[r],
                recv_sem=rs_recv_sems.at[r],
                device_id=(peers[r][0],),
                device_id_type=pl.DeviceIdType.MESH,
            )
            recv.wait_recv()
            red = red + stage_ref[pl.ds(r * CH, CH)].astype(jnp.float32)

        res_ref[pl.ds(my_off3, CH)] = red.astype(jnp.bfloat16)
        ag_sends = []
        for r in range(7):
            rdma = pltpu.make_async_remote_copy(
                src_ref=res_ref.at[pl.ds(my_off3, CH)],
                dst_ref=res_ref.at[pl.ds(my_off3, CH)],
                send_sem=ag_send_sems.at[r],
                recv_sem=ag_recv_sems.at[r],
                device_id=(peers[r][0],),
                device_id_type=pl.DeviceIdType.MESH,
            )
            rdma.start()
            ag_sends.append(rdma)

        for r in range(7):
            dev, off3 = peers[r]
            recv = pltpu.make_async_remote_copy(
                src_ref=res_ref.at[pl.ds(off3, CH)],
                dst_ref=res_ref.at[pl.ds(off3, CH)],
                send_sem=ag_send_sems.at[r],
                recv_sem=ag_recv_sems.at[r],
                device_id=(dev,),
                device_id_type=pl.DeviceIdType.MESH,
            )
            recv.wait_recv()

        for rdma in sends + ag_sends:
            rdma.wait_send()

        out_ref[...] = res_ref[...].astype(jnp.float32).reshape(B, S, OUT_N)

    return pl.pallas_call(
        body,
        out_shape=jax.ShapeDtypeStruct((B, S, OUT_N), jnp.float32),
        in_specs=[pl.BlockSpec(memory_space=pltpu.VMEM)] * 3,
        out_specs=pl.BlockSpec(memory_space=pltpu.VMEM),
        scratch_shapes=[
            pltpu.VMEM((M, OUT_N), jnp.bfloat16),
            pltpu.VMEM((7 * CH, OUT_N), jnp.bfloat16),
            pltpu.VMEM((M, OUT_N), jnp.bfloat16),
            pltpu.VMEM((1024, OUT_N), jnp.float32),
            pltpu.SemaphoreType.DMA((7,)),
            pltpu.SemaphoreType.DMA((7,)),
            pltpu.SemaphoreType.DMA((7,)),
            pltpu.SemaphoreType.DMA((7,)),
        ],
        compiler_params=pltpu.CompilerParams(collective_id=0),
    )(x, k, Wp)


# device time: 26639 ns/iter; 6.7987x vs baseline; 1.0627x over previous
import jax
import jax.numpy as jnp
from jax import lax
from jax.experimental import pallas as pl
from jax.experimental.pallas import tpu as pltpu

N_DEV = 8
B, S, C_LOC, T = 4, 512, 256, 4
OUT_N = 256
M = B * S
CH = 256
HALF = 128
N_PEER = 7

_DELTA_GROUPS = [
    [(1, 0, 0), (1, 0, 1)],
    [(1, 1, 0), (1, 1, 1)],
    [(0, 1, 0), (0, 1, 1)],
    [(0, 0, 1)],
]
_DELTAS = [d for g in _DELTA_GROUPS for d in g]


def kernel(x, k, Wp):
    def body(x_ref, k_ref, wp_ref, out_ref, sendbuf_ref, stage_ref, res_ref,
             near_ref, rs_send_sems, rs_recv_sems, ag_send_sems,
             ag_recv_sems):
        my = lax.axis_index("i")
        p = lax.rem(my, 4)
        bz = my // 4
        bx = jnp.bitwise_xor(p & 1, (p >> 1) & 1)
        by = (p >> 1) & 1

        def peer(delta):
            dx, dy, dz = delta
            tx = jnp.bitwise_xor(bx, dx)
            ty = jnp.bitwise_xor(by, dy)
            tz = jnp.bitwise_xor(bz, dz)
            dev = tz * 4 + ty * 2 + jnp.bitwise_xor(tx, ty)
            off3 = tx * 1024 + ty * 512 + tz * CH
            return dev, off3

        peers = [peer(d) for d in _DELTAS]
        my_off3 = bx * 1024 + by * 512 + bz * CH

        barrier_sem = pltpu.get_barrier_semaphore()
        for dev, _ in peers:
            pl.semaphore_signal(
                barrier_sem,
                inc=1,
                device_id=(dev,),
                device_id_type=pl.DeviceIdType.MESH,
            )
        pl.semaphore_wait(barrier_sem, N_DEV - 1)

        wp_bf16 = wp_ref[...].astype(jnp.bfloat16)

        def compute_quarter(b_off):
            xx = x_ref[pl.ds(b_off, 1)]
            acc = xx * k_ref[T - 1].reshape(1, 1, C_LOC)
            for lag in range(1, T):
                tap = k_ref[T - 1 - lag].reshape(1, 1, C_LOC)
                shifted = jnp.concatenate(
                    [jnp.zeros((1, lag, C_LOC), jnp.float32),
                     xx[:, : S - lag, :]],
                    axis=1,
                )
                acc = acc + shifted * tap
            a = acc * jax.nn.sigmoid(acc)
            return jnp.dot(
                a.reshape(512, C_LOC).astype(jnp.bfloat16),
                wp_bf16,
                preferred_element_type=jnp.float32,
            )

        def rs_send(r, h):
            dev, off3 = peers[r]
            rdma = pltpu.make_async_remote_copy(
                src_ref=sendbuf_ref.at[pl.ds(off3 + h * HALF, HALF)],
                dst_ref=stage_ref.at[pl.ds(h * 896 + r * HALF, HALF)],
                send_sem=rs_send_sems.at[h * N_PEER + r],
                recv_sem=rs_recv_sems.at[h * N_PEER + r],
                device_id=(dev,),
                device_id_type=pl.DeviceIdType.MESH,
            )
            rdma.start()
            return rdma

        sends = []
        r = 0
        for gi, group in enumerate(_DELTA_GROUPS):
            dx, dy = group[0][0], group[0][1]
            b_off = (jnp.bitwise_xor(bx, dx) * 2
                     + jnp.bitwise_xor(by, dy))
            q = compute_quarter(b_off)
            if gi == 3:
                near_ref[...] = q
            sendbuf_ref[pl.ds(b_off * 512, 512)] = q.astype(jnp.bfloat16)
            for _ in group:
                sends.append(rs_send(r, 0))
                sends.append(rs_send(r, 1))
                r += 1

        ag_sends = []
        for h in (0, 1):
            for r in range(N_PEER):
                recv = pltpu.make_async_remote_copy(
                    src_ref=sendbuf_ref.at[pl.ds(0, HALF)],
                    dst_ref=stage_ref.at[pl.ds(h * 896 + r * HALF, HALF)],
                    send_sem=rs_send_sems.at[h * N_PEER + r],
                    recv_sem=rs_recv_sems.at[h * N_PEER + r],
                    device_id=(peers[r][0],),
                    device_id_type=pl.DeviceIdType.MESH,
                )
                recv.wait_recv()
            red = near_ref[pl.ds(bz * CH + h * HALF, HALF)]
            for r in range(N_PEER):
                red = red + stage_ref[
                    pl.ds(h * 896 + r * HALF, HALF)
                ].astype(jnp.float32)
            res_ref[pl.ds(my_off3 + h * HALF, HALF)] = red.astype(
                jnp.bfloat16
            )
            for r in range(N_PEER):
                rdma = pltpu.make_async_remote_copy(
                    src_ref=res_ref.at[pl.ds(my_off3 + h * HALF, HALF)],
                    dst_ref=res_ref.at[pl.ds(my_off3 + h * HALF, HALF)],
                    send_sem=ag_send_sems.at[h * N_PEER + r],
                    recv_sem=ag_recv_sems.at[h * N_PEER + r],
                    device_id=(peers[r][0],),
                    device_id_type=pl.DeviceIdType.MESH,
                )
                rdma.start()
                ag_sends.append(rdma)

        for h in (0, 1):
            for r in range(N_PEER):
                dev, off3 = peers[r]
                recv = pltpu.make_async_remote_copy(
                    src_ref=res_ref.at[pl.ds(off3 + h * HALF, HALF)],
                    dst_ref=res_ref.at[pl.ds(off3 + h * HALF, HALF)],
                    send_sem=ag_send_sems.at[h * N_PEER + r],
                    recv_sem=ag_recv_sems.at[h * N_PEER + r],
                    device_id=(dev,),
                    device_id_type=pl.DeviceIdType.MESH,
                )
                recv.wait_recv()

        for rdma in sends + ag_sends:
            rdma.wait_send()

        out_ref[...] = res_ref[...].astype(jnp.float32).reshape(B, S, OUT_N)

    return pl.pallas_call(
        body,
        out_shape=jax.ShapeDtypeStruct((B, S, OUT_N), jnp.float32),
        in_specs=[pl.BlockSpec(memory_space=pltpu.VMEM)] * 3,
        out_specs=pl.BlockSpec(memory_space=pltpu.VMEM),
        scratch_shapes=[
            pltpu.VMEM((M, OUT_N), jnp.bfloat16),
            pltpu.VMEM((14 * HALF, OUT_N), jnp.bfloat16),
            pltpu.VMEM((M, OUT_N), jnp.bfloat16),
            pltpu.VMEM((512, OUT_N), jnp.float32),
            pltpu.SemaphoreType.DMA((14,)),
            pltpu.SemaphoreType.DMA((14,)),
            pltpu.SemaphoreType.DMA((14,)),
            pltpu.SemaphoreType.DMA((14,)),
        ],
        compiler_params=pltpu.CompilerParams(collective_id=0),
    )(x, k, Wp)


# device time: 26464 ns/iter; 6.8436x vs baseline; 1.0066x over previous
import jax
import jax.numpy as jnp
from jax import lax
from jax.experimental import pallas as pl
from jax.experimental.pallas import tpu as pltpu

N_DEV = 8
B, S, C_LOC, T = 4, 512, 256, 4
OUT_N = 256
M = B * S
CH = 256
HALF = 128
N_PEER = 7

_DELTA_GROUPS = [
    [(1, 0, 0), (1, 0, 1)],
    [(1, 1, 0), (1, 1, 1)],
    [(0, 1, 0), (0, 1, 1)],
    [(0, 0, 1)],
]
_DELTAS = [d for g in _DELTA_GROUPS for d in g]


def kernel(x, k, Wp):
    def body(x_ref, k_ref, wp_ref, out_ref, sendbuf_ref, stage_ref, res_ref,
             near_ref, rs_send_sems, rs_recv_sems, ag_send_sems,
             ag_recv_sems):
        my = lax.axis_index("i")
        p = lax.rem(my, 4)
        bz = my // 4
        bx = jnp.bitwise_xor(p & 1, (p >> 1) & 1)
        by = (p >> 1) & 1

        def peer(delta):
            dx, dy, dz = delta
            tx = jnp.bitwise_xor(bx, dx)
            ty = jnp.bitwise_xor(by, dy)
            tz = jnp.bitwise_xor(bz, dz)
            dev = tz * 4 + ty * 2 + jnp.bitwise_xor(tx, ty)
            off3 = tx * 1024 + ty * 512 + tz * CH
            return dev, off3

        peers = [peer(d) for d in _DELTAS]
        my_off3 = bx * 1024 + by * 512 + bz * CH

        barrier_sem = pltpu.get_barrier_semaphore()
        for dev, _ in peers:
            pl.semaphore_signal(
                barrier_sem,
                inc=1,
                device_id=(dev,),
                device_id_type=pl.DeviceIdType.MESH,
            )
        pl.semaphore_wait(barrier_sem, N_DEV - 1)

        wp_bf16 = wp_ref[...].astype(jnp.bfloat16)

        def compute_quarter(b_off):
            xx = x_ref[pl.ds(b_off, 1)]
            acc = xx * k_ref[T - 1].reshape(1, 1, C_LOC)
            for lag in range(1, T):
                tap = k_ref[T - 1 - lag].reshape(1, 1, C_LOC)
                shifted = jnp.concatenate(
                    [jnp.zeros((1, lag, C_LOC), jnp.float32),
                     xx[:, : S - lag, :]],
                    axis=1,
                )
                acc = acc + shifted * tap
            a = acc * jax.nn.sigmoid(acc)
            return jnp.dot(
                a.reshape(512, C_LOC).astype(jnp.bfloat16),
                wp_bf16,
                preferred_element_type=jnp.float32,
            )

        def rs_send(r, h):
            dev, off3 = peers[r]
            rdma = pltpu.make_async_remote_copy(
                src_ref=sendbuf_ref.at[pl.ds(off3 + h * HALF, HALF)],
                dst_ref=stage_ref.at[pl.ds(h * 896 + r * HALF, HALF)],
                send_sem=rs_send_sems.at[h * N_PEER + r],
                recv_sem=rs_recv_sems.at[h * N_PEER + r],
                device_id=(dev,),
                device_id_type=pl.DeviceIdType.MESH,
            )
            rdma.start()
            return rdma

        sends = []
        r = 0
        for gi, group in enumerate(_DELTA_GROUPS):
            dx, dy = group[0][0], group[0][1]
            b_off = (jnp.bitwise_xor(bx, dx) * 2
                     + jnp.bitwise_xor(by, dy))
            q = compute_quarter(b_off)
            if gi == 3:
                near_ref[...] = q
            sendbuf_ref[pl.ds(b_off * 512, 512)] = q.astype(jnp.bfloat16)
            for _ in group:
                sends.append(rs_send(r, 0))
                sends.append(rs_send(r, 1))
                r += 1

        ag_sends = []
        for h in (0, 1):
            for r in range(N_PEER):
                recv = pltpu.make_async_remote_copy(
                    src_ref=sendbuf_ref.at[pl.ds(0, HALF)],
                    dst_ref=stage_ref.at[pl.ds(h * 896 + r * HALF, HALF)],
                    send_sem=rs_send_sems.at[h * N_PEER + r],
                    recv_sem=rs_recv_sems.at[h * N_PEER + r],
                    device_id=(peers[r][0],),
                    device_id_type=pl.DeviceIdType.MESH,
                )
                recv.wait_recv()
            red = near_ref[pl.ds(bz * CH + h * HALF, HALF)]
            for r in range(N_PEER):
                red = red + stage_ref[
                    pl.ds(h * 896 + r * HALF, HALF)
                ].astype(jnp.float32)
            res_ref[pl.ds(my_off3 + h * HALF, HALF)] = red.astype(
                jnp.bfloat16
            )
            out_ref[bx * 2 + by, pl.ds(bz * CH + h * HALF, HALF), :] = red
            for r in range(N_PEER):
                rdma = pltpu.make_async_remote_copy(
                    src_ref=res_ref.at[pl.ds(my_off3 + h * HALF, HALF)],
                    dst_ref=res_ref.at[pl.ds(my_off3 + h * HALF, HALF)],
                    send_sem=ag_send_sems.at[h * N_PEER + r],
                    recv_sem=ag_recv_sems.at[h * N_PEER + r],
                    device_id=(peers[r][0],),
                    device_id_type=pl.DeviceIdType.MESH,
                )
                rdma.start()
                ag_sends.append(rdma)

        for h in (0, 1):
            for r in range(N_PEER):
                dev, off3 = peers[r]
                recv = pltpu.make_async_remote_copy(
                    src_ref=res_ref.at[pl.ds(off3 + h * HALF, HALF)],
                    dst_ref=res_ref.at[pl.ds(off3 + h * HALF, HALF)],
                    send_sem=ag_send_sems.at[h * N_PEER + r],
                    recv_sem=ag_recv_sems.at[h * N_PEER + r],
                    device_id=(dev,),
                    device_id_type=pl.DeviceIdType.MESH,
                )
                recv.wait_recv()
                dx, dy, dz = _DELTAS[r]
                b_idx = (jnp.bitwise_xor(bx, dx) * 2
                         + jnp.bitwise_xor(by, dy))
                s_off = jnp.bitwise_xor(bz, dz) * CH + h * HALF
                out_ref[b_idx, pl.ds(s_off, HALF), :] = res_ref[
                    pl.ds(off3 + h * HALF, HALF)
                ].astype(jnp.float32)

        for rdma in sends + ag_sends:
            rdma.wait_send()

    return pl.pallas_call(
        body,
        out_shape=jax.ShapeDtypeStruct((B, S, OUT_N), jnp.float32),
        in_specs=[pl.BlockSpec(memory_space=pltpu.VMEM)] * 3,
        out_specs=pl.BlockSpec(memory_space=pltpu.VMEM),
        scratch_shapes=[
            pltpu.VMEM((M, OUT_N), jnp.bfloat16),
            pltpu.VMEM((14 * HALF, OUT_N), jnp.bfloat16),
            pltpu.VMEM((M, OUT_N), jnp.bfloat16),
            pltpu.VMEM((512, OUT_N), jnp.float32),
            pltpu.SemaphoreType.DMA((14,)),
            pltpu.SemaphoreType.DMA((14,)),
            pltpu.SemaphoreType.DMA((14,)),
            pltpu.SemaphoreType.DMA((14,)),
        ],
        compiler_params=pltpu.CompilerParams(collective_id=0),
    )(x, k, Wp)


# device time: 25964 ns/iter; 6.9754x vs baseline; 1.0193x over previous
import jax
import jax.numpy as jnp
from jax import lax
from jax.experimental import pallas as pl
from jax.experimental.pallas import tpu as pltpu

N_DEV = 8
B, S, C_LOC, T = 4, 512, 256, 4
OUT_N = 256
M = B * S
CH = 256
HALF = 128
N_PEER = 7

_DELTA_GROUPS = [
    [(1, 0, 0), (1, 0, 1)],
    [(1, 1, 0), (1, 1, 1)],
    [(0, 1, 0), (0, 1, 1)],
    [(0, 0, 1)],
]
_DELTAS = [d for g in _DELTA_GROUPS for d in g]


def kernel(x, k, Wp):
    def body(x_ref, k_ref, wp_ref, out_ref, sendbuf_ref, stage_ref, res_ref,
             near_ref, rs_send_sems, rs_recv_sems, ag_send_sems,
             ag_recv_sems):
        my = lax.axis_index("i")
        p = lax.rem(my, 4)
        bz = my // 4
        bx = jnp.bitwise_xor(p & 1, (p >> 1) & 1)
        by = (p >> 1) & 1

        def peer(delta):
            dx, dy, dz = delta
            tx = jnp.bitwise_xor(bx, dx)
            ty = jnp.bitwise_xor(by, dy)
            tz = jnp.bitwise_xor(bz, dz)
            dev = tz * 4 + ty * 2 + jnp.bitwise_xor(tx, ty)
            off3 = tx * 1024 + ty * 512 + tz * CH
            return dev, off3

        peers = [peer(d) for d in _DELTAS]
        my_off3 = bx * 1024 + by * 512 + bz * CH

        barrier_sem = pltpu.get_barrier_semaphore()
        for dev, _ in peers:
            pl.semaphore_signal(
                barrier_sem,
                inc=1,
                device_id=(dev,),
                device_id_type=pl.DeviceIdType.MESH,
            )

        wp_bf16 = wp_ref[...].astype(jnp.bfloat16)

        def compute_quarter(b_off):
            xx = x_ref[pl.ds(b_off, 1)]
            acc = xx * k_ref[T - 1].reshape(1, 1, C_LOC)
            for lag in range(1, T):
                tap = k_ref[T - 1 - lag].reshape(1, 1, C_LOC)
                shifted = jnp.concatenate(
                    [jnp.zeros((1, lag, C_LOC), jnp.float32),
                     xx[:, : S - lag, :]],
                    axis=1,
                )
                acc = acc + shifted * tap
            a = acc * jax.nn.sigmoid(acc)
            return jnp.dot(
                a.reshape(512, C_LOC).astype(jnp.bfloat16),
                wp_bf16,
                preferred_element_type=jnp.float32,
            )

        def rs_send(r, h):
            dev, off3 = peers[r]
            rdma = pltpu.make_async_remote_copy(
                src_ref=sendbuf_ref.at[pl.ds(off3 + h * HALF, HALF)],
                dst_ref=stage_ref.at[pl.ds(h * 896 + r * HALF, HALF)],
                send_sem=rs_send_sems.at[h * N_PEER + r],
                recv_sem=rs_recv_sems.at[h * N_PEER + r],
                device_id=(dev,),
                device_id_type=pl.DeviceIdType.MESH,
            )
            rdma.start()
            return rdma

        sends = []
        r = 0
        for gi, group in enumerate(_DELTA_GROUPS):
            dx, dy = group[0][0], group[0][1]
            b_off = (jnp.bitwise_xor(bx, dx) * 2
                     + jnp.bitwise_xor(by, dy))
            q = compute_quarter(b_off)
            if gi == 0:
                pl.semaphore_wait(barrier_sem, N_DEV - 1)
            if gi == 3:
                near_ref[...] = q
            sendbuf_ref[pl.ds(b_off * 512, 512)] = q.astype(jnp.bfloat16)
            for _ in group:
                sends.append(rs_send(r, 0))
                sends.append(rs_send(r, 1))
                r += 1

        ag_sends = []
        for h in (0, 1):
            red = near_ref[pl.ds(bz * CH + h * HALF, HALF)]
            for r in range(N_PEER):
                recv = pltpu.make_async_remote_copy(
                    src_ref=sendbuf_ref.at[pl.ds(0, HALF)],
                    dst_ref=stage_ref.at[pl.ds(h * 896 + r * HALF, HALF)],
                    send_sem=rs_send_sems.at[h * N_PEER + r],
                    recv_sem=rs_recv_sems.at[h * N_PEER + r],
                    device_id=(peers[r][0],),
                    device_id_type=pl.DeviceIdType.MESH,
                )
                recv.wait_recv()
                red = red + stage_ref[
                    pl.ds(h * 896 + r * HALF, HALF)
                ].astype(jnp.float32)
            res_ref[pl.ds(my_off3 + h * HALF, HALF)] = red.astype(
                jnp.bfloat16
            )
            out_ref[bx * 2 + by, pl.ds(bz * CH + h * HALF, HALF), :] = red
            for r in range(N_PEER):
                rdma = pltpu.make_async_remote_copy(
                    src_ref=res_ref.at[pl.ds(my_off3 + h * HALF, HALF)],
                    dst_ref=res_ref.at[pl.ds(my_off3 + h * HALF, HALF)],
                    send_sem=ag_send_sems.at[h * N_PEER + r],
                    recv_sem=ag_recv_sems.at[h * N_PEER + r],
                    device_id=(peers[r][0],),
                    device_id_type=pl.DeviceIdType.MESH,
                )
                rdma.start()
                ag_sends.append(rdma)

        for h in (0, 1):
            for r in range(N_PEER):
                dev, off3 = peers[r]
                recv = pltpu.make_async_remote_copy(
                    src_ref=res_ref.at[pl.ds(off3 + h * HALF, HALF)],
                    dst_ref=res_ref.at[pl.ds(off3 + h * HALF, HALF)],
                    send_sem=ag_send_sems.at[h * N_PEER + r],
                    recv_sem=ag_recv_sems.at[h * N_PEER + r],
                    device_id=(dev,),
                    device_id_type=pl.DeviceIdType.MESH,
                )
                recv.wait_recv()
                dx, dy, dz = _DELTAS[r]
                b_idx = (jnp.bitwise_xor(bx, dx) * 2
                         + jnp.bitwise_xor(by, dy))
                s_off = jnp.bitwise_xor(bz, dz) * CH + h * HALF
                out_ref[b_idx, pl.ds(s_off, HALF), :] = res_ref[
                    pl.ds(off3 + h * HALF, HALF)
                ].astype(jnp.float32)

        for rdma in sends + ag_sends:
            rdma.wait_send()

    return pl.pallas_call(
        body,
        out_shape=jax.ShapeDtypeStruct((B, S, OUT_N), jnp.float32),
        in_specs=[pl.BlockSpec(memory_space=pltpu.VMEM)] * 3,
        out_specs=pl.BlockSpec(memory_space=pltpu.VMEM),
        scratch_shapes=[
            pltpu.VMEM((M, OUT_N), jnp.bfloat16),
            pltpu.VMEM((14 * HALF, OUT_N), jnp.bfloat16),
            pltpu.VMEM((M, OUT_N), jnp.bfloat16),
            pltpu.VMEM((512, OUT_N), jnp.float32),
            pltpu.SemaphoreType.DMA((14,)),
            pltpu.SemaphoreType.DMA((14,)),
            pltpu.SemaphoreType.DMA((14,)),
            pltpu.SemaphoreType.DMA((14,)),
        ],
        compiler_params=pltpu.CompilerParams(collective_id=0),
    )(x, k, Wp)
